# baseline (speedup 1.0000x reference)
import functools

import jax
import jax.numpy as jnp
from jax.experimental import pallas as pl
from jax.experimental.pallas import tpu as pltpu

_B, _S, _D = 16, 4096, 1024
_BS = 2048
_NBJ = _S // _BS


def _tc_body(nblk_ref, lens_ref, hs_ref, o_ref, acc_ref):
    i = pl.program_id(0)
    j = pl.program_id(1)

    @pl.when(j == 0)
    def _():
        acc_ref[...] = jnp.zeros_like(acc_ref)

    L = lens_ref[i]
    active = j < nblk_ref[i]
    full = active & (j > 0) & ((j + 1) * _BS - 1 <= L)

    @pl.when(full)
    def _():
        xs = hs_ref[...].reshape(_BS // 8, 8, _D)
        acc_ref[...] += jnp.sum(xs, axis=0)

    @pl.when(active & jnp.logical_not(full))
    def _():
        pos = jax.lax.broadcasted_iota(jnp.int32, (_BS, 1), 0) + j * _BS
        m = ((pos >= 1) & (pos <= L)).astype(jnp.float32)
        xs = (hs_ref[...] * m).reshape(_BS // 8, 8, _D)
        acc_ref[...] += jnp.sum(xs, axis=0)

    @pl.when(j == _NBJ - 1)
    def _():
        inv = 1.0 / L.astype(jnp.float32)
        o_ref[0] = jnp.sum(acc_ref[...], axis=0, keepdims=True) * inv


@jax.jit
def kernel(hidden_states, pivot_len_list):
    hs2 = hidden_states.reshape(_B * _S, _D)
    nblk = pivot_len_list // _BS + 1
    grid_spec = pltpu.PrefetchScalarGridSpec(
        num_scalar_prefetch=2,
        grid=(_B, _NBJ),
        in_specs=[
            pl.BlockSpec(
                (_BS, _D),
                lambda i, j, nblk_ref, lens_ref: (
                    i * _NBJ + jnp.minimum(j, nblk_ref[i] - 1), 0),
            ),
        ],
        out_specs=pl.BlockSpec(
            (1, 1, _D), lambda i, j, nblk_ref, lens_ref: (i, 0, 0)),
        scratch_shapes=[pltpu.VMEM((8, _D), jnp.float32)],
    )
    pool = pl.pallas_call(
        _tc_body,
        grid_spec=grid_spec,
        out_shape=jax.ShapeDtypeStruct((_B, 1, _D), jnp.float32),
    )
    return pool(nblk, pivot_len_list, hs2).reshape(_B, _D)


# speedup vs baseline: 1.0012x; 1.0012x over previous
import functools

import jax
import jax.numpy as jnp
from jax import lax
from jax.experimental import pallas as pl
from jax.experimental.pallas import tpu as pltpu
from jax.experimental.pallas import tpu_sc as plsc

_B, _S, _D = 16, 4096, 1024
_K = 5
_LANES = 16
_T = 16
_RING = 4
_NW = 32
_GPD = _D // _LANES

_BS = 1024
_NBJ = _S // _BS
_BT = _B - _K


def _sc_body(hs2, lens, psum, len_v, bufs, tbuf, acc, rbuf, obuf, stage_sh,
             semc, semt):
    c = lax.axis_index("c")
    s = lax.axis_index("s")
    w = s * 2 + c

    pltpu.sync_copy(lens, len_v.at[pl.ds(0, _B)])

    zeros = jnp.zeros((_LANES,), jnp.float32)

    def zacc(k, carry):
        acc[0, pl.ds(k * _LANES, _LANES)] = zeros
        return carry

    lax.fori_loop(0, _K * _GPD, zacc, 0)

    def vacc(p, i):
        def vbody(v, carry):
            q = v * _LANES
            sl = pl.ds(q, _LANES)
            xs = [bufs[p, r, sl] for r in range(_T)]
            while len(xs) > 1:
                xs = [a + b for a, b in zip(xs[0::2], xs[1::2])]
            plsc.addupdate(acc.at[0, pl.ds(i * _D + q, _LANES)], xs[0])
            return carry

        lax.fori_loop(0, _GPD, vbody, 0)

    for i in range(_K):
        L = len_v[pl.ds(i, _LANES)][0]
        U = (L + 8) // 8
        start = 8 * ((w * U) // _NW)
        rend = 8 * (((w + 1) * U) // _NW)
        cnt = jnp.minimum(rend, L + 1) - start
        nf = cnt // _T
        rem = cnt - nf * _T
        rowbase = i * _S

        tst_raw = start + nf * _T
        tst = jnp.minimum(tst_raw, _S - _T)
        tlo = tst_raw - tst

        @pl.when(rem > 0)
        def _():
            pltpu.async_copy(
                hs2.at[pl.ds(rowbase + tst, _T), :], tbuf, semt)

        def rloop(k, carry):
            @pl.when(k >= _RING)
            def _():
                pltpu.make_async_copy(
                    hs2.at[pl.ds(rowbase, _T), :], bufs.at[0], semc).wait()
                vacc(lax.rem(k - _RING, _RING), i)

            @pl.when(k < nf)
            def _():
                pltpu.async_copy(
                    hs2.at[pl.ds(rowbase + start + k * _T, _T), :],
                    bufs.at[lax.rem(k, _RING)], semc)

            return carry

        lax.fori_loop(0, nf + _RING, rloop, 0)

        @pl.when(rem > 0)
        def _():
            pltpu.make_async_copy(
                hs2.at[pl.ds(rowbase, _T), :], tbuf, semt).wait()

        def tbody(r, carry):
            for v in range(_GPD):
                plsc.addupdate(
                    acc.at[0, pl.ds(i * _D + v * _LANES, _LANES)],
                    tbuf[r, pl.ds(v * _LANES, _LANES)])
            return carry

        lax.fori_loop(tlo, tlo + rem, tbody, 0)

        @pl.when(w == 0)
        def _():
            pltpu.sync_copy(hs2.at[pl.ds(rowbase, _T), :], tbuf)

            def sbody(v, carry):
                q = v * _LANES
                plsc.addupdate(
                    acc.at[0, pl.ds(i * _D + q, _LANES)],
                    0.0 - tbuf[0, pl.ds(q, _LANES)])
                return carry

            lax.fori_loop(0, _GPD, sbody, 0)

    pltpu.sync_copy(acc, stage_sh.at[pl.ds(s, 1)])
    plsc.subcore_barrier()

    @pl.when(s < _K)
    def _():
        pltpu.sync_copy(
            stage_sh.at[pl.ds(0, 16), pl.ds(s * _D, _D)], rbuf)

        def redbody(v, carry):
            sl = pl.ds(v * _LANES, _LANES)
            x = rbuf[0, sl]
            for t in range(1, 16):
                x = x + rbuf[t, sl]
            obuf[0, sl] = x
            return carry

        lax.fori_loop(0, _GPD, redbody, 0)
        pltpu.sync_copy(obuf, psum.at[c, pl.ds(s, 1)])


def _tc_body(nblk_ref, lens_ref, hs_ref, o_ref, acc_ref):
    i = pl.program_id(0)
    j = pl.program_id(1)

    @pl.when(j == 0)
    def _():
        acc_ref[...] = jnp.zeros_like(acc_ref)

    L = lens_ref[i + _K]
    active = j < nblk_ref[i + _K]
    full = active & (j > 0) & ((j + 1) * _BS - 1 <= L)

    @pl.when(full)
    def _():
        xs = hs_ref[...].reshape(_BS // 8, 8, _D)
        acc_ref[...] += jnp.sum(xs, axis=0)

    @pl.when(active & jnp.logical_not(full))
    def _():
        pos = jax.lax.broadcasted_iota(jnp.int32, (_BS, 1), 0) + j * _BS
        m = ((pos >= 1) & (pos <= L)).astype(jnp.float32)
        xs = (hs_ref[...] * m).reshape(_BS // 8, 8, _D)
        acc_ref[...] += jnp.sum(xs, axis=0)

    @pl.when(j == _NBJ - 1)
    def _():
        inv = 1.0 / L.astype(jnp.float32)
        o_ref[0] = jnp.sum(acc_ref[...], axis=0, keepdims=True) * inv


def _combine_body(psum_ref, lens_ref, o_ref):
    inv = 1.0 / lens_ref[...].astype(jnp.float32)
    o_ref[...] = (psum_ref[0] + psum_ref[1]) * inv


@jax.jit
def kernel(hidden_states, pivot_len_list):
    hs2 = hidden_states.reshape(_B * _S, _D)
    nblk = pivot_len_list // _BS + 1

    mesh = plsc.VectorSubcoreMesh(core_axis_name="c", subcore_axis_name="s")
    sc_pool = functools.partial(
        pl.kernel,
        out_type=jax.ShapeDtypeStruct((2, _K, _D), jnp.float32),
        mesh=mesh,
        scratch_types=[
            pltpu.VMEM((2 * _LANES,), jnp.int32),
            pltpu.VMEM((_RING, _T, _D), jnp.float32),
            pltpu.VMEM((_T, _D), jnp.float32),
            pltpu.VMEM((1, _K * _D), jnp.float32),
            pltpu.VMEM((16, _D), jnp.float32),
            pltpu.VMEM((1, _D), jnp.float32),
            pltpu.VMEM_SHARED((16, _K * _D), jnp.float32),
            pltpu.SemaphoreType.DMA,
            pltpu.SemaphoreType.DMA,
        ],
    )(_sc_body)
    psum = sc_pool(hs2, pivot_len_list)

    grid_spec = pltpu.PrefetchScalarGridSpec(
        num_scalar_prefetch=2,
        grid=(_BT, _NBJ),
        in_specs=[
            pl.BlockSpec(
                (_BS, _D),
                lambda i, j, nblk_ref, lens_ref: (
                    (i + _K) * _NBJ
                    + jnp.minimum(j, nblk_ref[i + _K] - 1), 0),
            ),
        ],
        out_specs=pl.BlockSpec(
            (1, 1, _D), lambda i, j, nblk_ref, lens_ref: (i, 0, 0)),
        scratch_shapes=[pltpu.VMEM((8, _D), jnp.float32)],
    )
    otc = pl.pallas_call(
        _tc_body,
        grid_spec=grid_spec,
        out_shape=jax.ShapeDtypeStruct((_BT, 1, _D), jnp.float32),
    )(nblk, pivot_len_list, hs2)

    osc = pl.pallas_call(
        _combine_body,
        out_shape=jax.ShapeDtypeStruct((_K, _D), jnp.float32),
    )(psum, pivot_len_list[:_K].reshape(_K, 1))

    return jnp.concatenate([osc, otc.reshape(_BT, _D)], axis=0)


# speedup vs baseline: 1.1160x; 1.1147x over previous
import functools

import jax
import jax.numpy as jnp
from jax.experimental import pallas as pl
from jax.experimental.pallas import tpu as pltpu

_B, _S, _D = 16, 4096, 1024
_BS = 1024
_NBJ = _S // _BS


def _tc_body(nblk_ref, lens_ref, hs_ref, o_ref, acc_ref):
    i = pl.program_id(0)
    j = pl.program_id(1)

    @pl.when(j == 0)
    def _():
        acc_ref[...] = jnp.zeros_like(acc_ref)

    L = lens_ref[i]
    active = j < nblk_ref[i]
    full = active & (j > 0) & ((j + 1) * _BS - 1 <= L)

    @pl.when(full)
    def _():
        xs = hs_ref[...].reshape(_BS // 8, 8, _D)
        acc_ref[...] += jnp.sum(xs, axis=0)

    @pl.when(active & jnp.logical_not(full))
    def _():
        pos = jax.lax.broadcasted_iota(jnp.int32, (_BS, 1), 0) + j * _BS
        m = ((pos >= 1) & (pos <= L)).astype(jnp.float32)
        xs = (hs_ref[...] * m).reshape(_BS // 8, 8, _D)
        acc_ref[...] += jnp.sum(xs, axis=0)

    @pl.when(j == _NBJ - 1)
    def _():
        inv = 1.0 / L.astype(jnp.float32)
        o_ref[0] = jnp.sum(acc_ref[...], axis=0, keepdims=True) * inv


@jax.jit
def kernel(hidden_states, pivot_len_list):
    hs2 = hidden_states.reshape(_B * _S, _D)
    nblk = pivot_len_list // _BS + 1
    grid_spec = pltpu.PrefetchScalarGridSpec(
        num_scalar_prefetch=2,
        grid=(_B, _NBJ),
        in_specs=[
            pl.BlockSpec(
                (_BS, _D),
                lambda i, j, nblk_ref, lens_ref: (
                    i * _NBJ + jnp.minimum(j, nblk_ref[i] - 1), 0),
            ),
        ],
        out_specs=pl.BlockSpec(
            (1, 1, _D), lambda i, j, nblk_ref, lens_ref: (i, 0, 0)),
        scratch_shapes=[pltpu.VMEM((8, _D), jnp.float32)],
    )
    pool = pl.pallas_call(
        _tc_body,
        grid_spec=grid_spec,
        out_shape=jax.ShapeDtypeStruct((_B, 1, _D), jnp.float32),
    )
    return pool(nblk, pivot_len_list, hs2).reshape(_B, _D)
